# Initial kernel scaffold; baseline (speedup 1.0000x reference)
import jax
import jax.numpy as jnp
from jax.experimental import pallas as pl


def kernel(input_ids, segment_ids, word_emb, pos_emb, seg_emb):
    raise NotImplementedError("write your pallas kernel here")


# speedup vs baseline: 2.0168x; 2.0168x over previous
import functools

import jax
import jax.numpy as jnp
from jax import lax
from jax.experimental import pallas as pl
from jax.experimental.pallas import tpu as pltpu
from jax.experimental.pallas import tpu_sc as plsc

NC, NS = 2, 16
NW = NC * NS
G = 128
L = 16


@functools.lru_cache(maxsize=None)
def _build(R, D, S, NSEG):
    RW = R // NW

    def body(word_hbm, comb_hbm, wid_hbm, seg_hbm, out_hbm,
             widx_v, cidx_v, wbuf, cbuf, sem_w, sem_c):
        w = lax.axis_index("s") * NC + lax.axis_index("c")
        base = w * RW
        pltpu.sync_copy(wid_hbm.at[w], widx_v)
        pltpu.sync_copy(seg_hbm.at[w], cidx_v)

        def row_body(r, carry):
            tok0 = (base + r) * G
            for j in range(G // L):
                t = tok0 + j * L + lax.iota(jnp.int32, L)
                pos = lax.rem(t, S)
                cidx_v[r, pl.ds(j * L, L)] = (
                    pos * NSEG + cidx_v[r, pl.ds(j * L, L)])
            return carry
        lax.fori_loop(0, RW, row_body, 0)

        def g_body(g, carry):
            cp_w = pltpu.async_copy(word_hbm.at[widx_v.at[g]], wbuf, sem_w)
            cp_c = pltpu.async_copy(comb_hbm.at[cidx_v.at[g]], cbuf, sem_c)
            cp_w.wait()
            cp_c.wait()

            def t_body(t, c2):
                for j in range(D // L):
                    v = cbuf[t, pl.ds(j * L, L)]
                    plsc.addupdate(wbuf.at[t, pl.ds(j * L, L)], v)
                return c2
            lax.fori_loop(0, G, t_body, 0)
            pltpu.sync_copy(wbuf, out_hbm.at[base + g])
            return carry
        lax.fori_loop(0, RW, g_body, 0)

    mesh = plsc.VectorSubcoreMesh(
        core_axis_name="c", subcore_axis_name="s",
        num_cores=NC, num_subcores=NS)
    return pl.kernel(
        body,
        out_type=jax.ShapeDtypeStruct((R, G, D), jnp.float32),
        mesh=mesh,
        compiler_params=pltpu.CompilerParams(use_tc_tiling_on_sc=False),
        scratch_types=[
            pltpu.VMEM((RW, G), jnp.int32),
            pltpu.VMEM((RW, G), jnp.int32),
            pltpu.VMEM((G, D), jnp.float32),
            pltpu.VMEM((G, D), jnp.float32),
            pltpu.SemaphoreType.DMA,
            pltpu.SemaphoreType.DMA,
        ],
    )


def kernel(input_ids, segment_ids, word_emb, pos_emb, seg_emb):
    b, s = input_ids.shape
    d = word_emb.shape[-1]
    nseg = seg_emb.shape[0]
    n = b * s
    ids2 = input_ids.astype(jnp.int32).reshape(NW, n // (G * NW), G)
    seg2 = segment_ids.astype(jnp.int32).reshape(NW, n // (G * NW), G)
    comb = (pos_emb[:, None, :] + seg_emb[None, :, :]).reshape(-1, d)
    out = _build(n // G, d, s, nseg)(word_emb, comb, ids2, seg2)
    return out.reshape(b, s, d)
